# baseline (speedup 1.0000x reference)
import functools

import jax
import jax.numpy as jnp
from jax import lax
from jax.experimental import pallas as pl
from jax.experimental.pallas import tpu as pltpu
from jax.experimental.pallas import tpu_sc as plsc

_LANES = 16


def _last_token_gather(x_hbm, idx_hbm, out_hbm, idx_v, rows_v, sem):
    B, C = out_hbm.shape
    cid = lax.axis_index("c")
    sid = lax.axis_index("s")

    @pl.when(jnp.logical_and(cid == 0, sid == 0))
    def _():
        pltpu.sync_copy(idx_hbm, idx_v)
        pltpu.async_copy(x_hbm.at[idx_v], rows_v, sem).wait()
        pltpu.sync_copy(rows_v.at[pl.ds(0, B)], out_hbm)


def kernel(x, lengths):
    B, T, C = x.shape
    x_flat = x.reshape(B * T, C)
    lane = jnp.arange(_LANES, dtype=jnp.int32)
    li = jnp.maximum(lengths.astype(jnp.int32) - 1, 0)
    idx = jnp.where(lane < B, jnp.pad(li, (0, _LANES - B)) + lane * T, 0)

    mesh = plsc.VectorSubcoreMesh(core_axis_name="c", subcore_axis_name="s")
    run = functools.partial(
        pl.kernel,
        out_type=jax.ShapeDtypeStruct((B, C), x.dtype),
        mesh=mesh,
        scratch_types=[
            pltpu.VMEM((_LANES,), jnp.int32),
            pltpu.VMEM((_LANES, C), x.dtype),
            pltpu.SemaphoreType.DMA,
        ],
    )(_last_token_gather)
    return run(x_flat, idx)


# speedup vs baseline: 1.0616x; 1.0616x over previous
import functools

import jax
import jax.numpy as jnp
from jax import lax
from jax.experimental import pallas as pl
from jax.experimental.pallas import tpu as pltpu
from jax.experimental.pallas import tpu_sc as plsc

_LANES = 16


def _last_token_gather(x_hbm, idx_hbm, out_hbm, idx_v, sem):
    B, C = out_hbm.shape
    cid = lax.axis_index("c")
    sid = lax.axis_index("s")

    @pl.when(jnp.logical_and(cid == 0, sid == 0))
    def _():
        pltpu.sync_copy(idx_hbm, idx_v)
        iv = idx_v[...]
        copies = []
        for b in range(B):
            ib = iv[b]
            copies.append(
                pltpu.make_async_copy(
                    x_hbm.at[pl.ds(ib, 1)], out_hbm.at[pl.ds(b, 1)], sem
                )
            )
        for cp in copies:
            cp.start()
        for cp in copies:
            cp.wait()


def kernel(x, lengths):
    B, T, C = x.shape
    x_flat = x.reshape(B * T, C)
    lane = jnp.arange(_LANES, dtype=jnp.int32)
    li = jnp.maximum(lengths.astype(jnp.int32) - 1, 0)
    idx = jnp.where(lane < B, jnp.pad(li, (0, _LANES - B)) + lane * T, 0)

    mesh = plsc.VectorSubcoreMesh(core_axis_name="c", subcore_axis_name="s")
    run = functools.partial(
        pl.kernel,
        out_type=jax.ShapeDtypeStruct((B, C), x.dtype),
        mesh=mesh,
        scratch_types=[
            pltpu.VMEM((_LANES,), jnp.int32),
            pltpu.SemaphoreType.DMA,
        ],
    )(_last_token_gather)
    return run(x_flat, idx)


# speedup vs baseline: 1.1710x; 1.1031x over previous
import functools

import jax
import jax.numpy as jnp
from jax import lax
from jax.experimental import pallas as pl
from jax.experimental.pallas import tpu as pltpu
from jax.experimental.pallas import tpu_sc as plsc

_LANES = 16


def _last_token_gather(x_hbm, idx_hbm, out_hbm, idx_s, sem):
    B, C = out_hbm.shape
    cid = lax.axis_index("c")

    @pl.when(cid == 0)
    def _():
        pltpu.sync_copy(idx_hbm, idx_s)
        copies = []
        for b in range(B):
            ib = idx_s[b]
            copies.append(
                pltpu.make_async_copy(
                    x_hbm.at[pl.ds(ib, 1)], out_hbm.at[pl.ds(b, 1)], sem
                )
            )
        for cp in copies:
            cp.start()
        for cp in copies:
            cp.wait()


def kernel(x, lengths):
    B, T, C = x.shape
    x_flat = x.reshape(B * T, C)
    lane = jnp.arange(_LANES, dtype=jnp.int32)
    li = jnp.maximum(lengths.astype(jnp.int32) - 1, 0)
    idx = jnp.where(lane < B, jnp.pad(li, (0, _LANES - B)) + lane * T, 0)

    mesh = plsc.ScalarSubcoreMesh(axis_name="c", num_cores=2)
    run = functools.partial(
        pl.kernel,
        out_type=jax.ShapeDtypeStruct((B, C), x.dtype),
        mesh=mesh,
        scratch_types=[
            pltpu.SMEM((_LANES,), jnp.int32),
            pltpu.SemaphoreType.DMA,
        ],
    )(_last_token_gather)
    return run(x_flat, idx)


# speedup vs baseline: 1.2593x; 1.0755x over previous
import functools

import jax
import jax.numpy as jnp
from jax import lax
from jax.experimental import pallas as pl
from jax.experimental.pallas import tpu as pltpu
from jax.experimental.pallas import tpu_sc as plsc

_LANES = 16


def _last_token_gather(x_hbm, idx_hbm, out_hbm, idx_s, sem):
    B, C = out_hbm.shape
    cid = lax.axis_index("c")

    @pl.when(cid == 0)
    def _():
        pltpu.sync_copy(idx_hbm, idx_s)
        copies = []
        for b in range(B):
            ib = idx_s[b]
            copies.append(
                pltpu.make_async_copy(
                    x_hbm.at[pl.ds(ib, 1)], out_hbm.at[pl.ds(b, 1)], sem
                )
            )
        for cp in copies:
            cp.start()
        for cp in copies:
            cp.wait()


def kernel(x, lengths):
    B, T, C = x.shape
    x_flat = x.reshape(B * T, C)
    lane = jnp.arange(_LANES, dtype=jnp.int32)
    li = jnp.maximum(lengths.astype(jnp.int32) - 1, 0)
    idx = jnp.where(lane < B, jnp.pad(li, (0, _LANES - B)) + lane * T, 0)

    mesh = plsc.ScalarSubcoreMesh(axis_name="c", num_cores=1)
    run = functools.partial(
        pl.kernel,
        out_type=jax.ShapeDtypeStruct((B, C), x.dtype),
        mesh=mesh,
        scratch_types=[
            pltpu.SMEM((_LANES,), jnp.int32),
            pltpu.SemaphoreType.DMA,
        ],
    )(_last_token_gather)
    return run(x_flat, idx)
